# baseline (speedup 1.0000x reference)
import functools

import jax
import jax.numpy as jnp
from jax import lax
from jax.experimental import pallas as pl
from jax.experimental.pallas import tpu as pltpu
from jax.experimental.pallas import tpu_sc as plsc

N = 10000
E = 320000
D = 128
G = 64
NT = 32
EPT = E // NT
K = 80
NCH = EPT // K
NPAD = 10240
RPT = NPAD // 16
BR = 2000
GRID = N // BR

_mesh = plsc.VectorSubcoreMesh(core_axis_name="c", subcore_axis_name="s")


@functools.partial(
    pl.kernel,
    mesh=_mesh,
    out_type=jax.ShapeDtypeStruct((2, 16 * NPAD), jnp.float32),
    scratch_types=[
        pltpu.VMEM((NCH, K), jnp.int32),
        pltpu.VMEM((K,), jnp.float32),
        pltpu.VMEM_SHARED((16 * NPAD,), jnp.float32),
    ],
)
def _deg_kernel(dst_hbm, ones_hbm, zeros_hbm, out_hbm, dst_v, ones_v, acc_sh):
    cid = lax.axis_index("c")
    sid = lax.axis_index("s")
    wid = cid * 16 + sid
    off = pl.multiple_of(sid * NPAD, 8)
    pltpu.sync_copy(dst_hbm.at[wid], dst_v)
    pltpu.sync_copy(ones_hbm, ones_v)
    pltpu.sync_copy(zeros_hbm.at[pl.ds(off, NPAD)],
                    acc_sh.at[pl.ds(off, NPAD)])
    plsc.subcore_barrier()

    def body(c, carry):
        pltpu.sync_copy(ones_v, acc_sh.at[dst_v.at[c]], add=True)
        return carry

    lax.fori_loop(0, NCH, body, 0)
    plsc.subcore_barrier()
    pltpu.sync_copy(acc_sh.at[pl.ds(off, NPAD)],
                    out_hbm.at[cid, pl.ds(off, NPAD)])


KE = 128
EPTP = 10240
NPH = 4
PC = EPTP // KE // NPH
PPAIR = PC // 2


@functools.partial(
    pl.kernel,
    mesh=_mesh,
    out_type=jax.ShapeDtypeStruct((2, NPAD, D), jnp.float32),
    scratch_types=[
        pltpu.VMEM((PC, KE), jnp.int32),
        pltpu.VMEM((PC, KE), jnp.int32),
        pltpu.VMEM((KE, D), jnp.float32),
        pltpu.VMEM((KE, D), jnp.float32),
        pltpu.VMEM_SHARED((NPAD, D), jnp.float32),
        pltpu.SemaphoreType.DMA,
        pltpu.SemaphoreType.DMA,
    ],
)
def _edge_kernel(table_hbm, src_hbm, dst_hbm, zeros_hbm, out_hbm,
                 src_v, dst_v, rows_a, rows_b, acc_sh, sem_a, sem_b):
    cid = lax.axis_index("c")
    sid = lax.axis_index("s")
    wid = cid * 16 + sid
    pltpu.sync_copy(zeros_hbm.at[pl.ds(sid * RPT, RPT)],
                    acc_sh.at[pl.ds(sid * RPT, RPT)])
    plsc.subcore_barrier()

    def start(c, buf, sem):
        pltpu.async_copy(table_hbm.at[src_v.at[c]], buf, sem)

    def wait(c, buf, sem):
        pltpu.make_async_copy(table_hbm.at[src_v.at[c]], buf, sem).wait()

    def phase(p, carry):
        pltpu.sync_copy(src_hbm.at[wid, p], src_v)
        pltpu.sync_copy(dst_hbm.at[wid, p], dst_v)
        start(0, rows_a, sem_a)

        def body(i, carry2):
            c0 = 2 * i
            c1 = c0 + 1
            start(c1, rows_b, sem_b)
            wait(c0, rows_a, sem_a)
            pltpu.sync_copy(rows_a, acc_sh.at[dst_v.at[c0]], add=True)

            @pl.when(i < PPAIR - 1)
            def _():
                start(c0 + 2, rows_a, sem_a)

            wait(c1, rows_b, sem_b)
            pltpu.sync_copy(rows_b, acc_sh.at[dst_v.at[c1]], add=True)
            return carry2

        lax.fori_loop(0, PPAIR, body, 0)
        return carry

    lax.fori_loop(0, NPH, phase, 0)
    plsc.subcore_barrier()
    pltpu.sync_copy(acc_sh.at[pl.ds(sid * RPT, RPT)],
                    out_hbm.at[cid, pl.ds(sid * RPT, RPT)])


_HI = jax.lax.Precision.HIGHEST


def _mm1_body(x_ref, w_ref, dg_ref, hs_ref, dinv_ref):
    deg = jnp.sum(dg_ref[...], axis=1, keepdims=True) + 1.0
    dinv = lax.rsqrt(deg)
    h = jnp.dot(x_ref[...], w_ref[...], preferred_element_type=jnp.float32,
                precision=_HI)
    hs_ref[...] = h * dinv
    dinv_ref[...] = dinv


def _mid_body(acc_ref, hs_ref, dinv_ref, b_ref, w_ref, out_ref):
    dinv = dinv_ref[...]
    m = (acc_ref[0] + acc_ref[1] + hs_ref[...]) * dinv + b_ref[...]
    m = jnp.maximum(m, 0.0)
    out_ref[...] = jnp.dot(m, w_ref[...], preferred_element_type=jnp.float32,
                           precision=_HI) * dinv


def _fin_body(acc_ref, hs_ref, dinv_ref, b_ref, wfc_ref, bfc_ref, batch_ref,
              out_ref, sums_ref, cnts_ref):
    i = pl.program_id(0)
    m = (acc_ref[0] + acc_ref[1] + hs_ref[...]) * dinv_ref[...] + b_ref[...]
    m = jnp.maximum(m, 0.0)
    y = jnp.sum(m * wfc_ref[...], axis=1, keepdims=True)
    oh = (batch_ref[...] == lax.broadcasted_iota(jnp.int32, (BR, G), 1))
    oh = oh.astype(jnp.float32)
    s = jnp.sum(oh * y, axis=0, keepdims=True)
    c = jnp.sum(oh, axis=0, keepdims=True)

    @pl.when(i == 0)
    def _():
        sums_ref[...] = s
        cnts_ref[...] = c

    @pl.when(i > 0)
    def _():
        sums_ref[...] += s
        cnts_ref[...] += c

    @pl.when(i == GRID - 1)
    def _():
        out_ref[...] = (sums_ref[...] / jnp.maximum(cnts_ref[...], 1.0)
                        + bfc_ref[...])


_mm1 = pl.pallas_call(
    _mm1_body,
    grid=(GRID,),
    in_specs=[
        pl.BlockSpec((BR, D), lambda i: (i, 0)),
        pl.BlockSpec((D, D), lambda i: (0, 0)),
        pl.BlockSpec((BR, NT), lambda i: (i, 0)),
    ],
    out_specs=[
        pl.BlockSpec((BR, D), lambda i: (i, 0)),
        pl.BlockSpec((BR, 1), lambda i: (i, 0)),
    ],
    out_shape=[
        jax.ShapeDtypeStruct((N, D), jnp.float32),
        jax.ShapeDtypeStruct((N, 1), jnp.float32),
    ],
)

_mid = pl.pallas_call(
    _mid_body,
    grid=(GRID,),
    in_specs=[
        pl.BlockSpec((2, BR, D), lambda i: (0, i, 0)),
        pl.BlockSpec((BR, D), lambda i: (i, 0)),
        pl.BlockSpec((BR, 1), lambda i: (i, 0)),
        pl.BlockSpec((1, D), lambda i: (0, 0)),
        pl.BlockSpec((D, D), lambda i: (0, 0)),
    ],
    out_specs=pl.BlockSpec((BR, D), lambda i: (i, 0)),
    out_shape=jax.ShapeDtypeStruct((N, D), jnp.float32),
)

_fin = pl.pallas_call(
    _fin_body,
    grid=(GRID,),
    in_specs=[
        pl.BlockSpec((2, BR, D), lambda i: (0, i, 0)),
        pl.BlockSpec((BR, D), lambda i: (i, 0)),
        pl.BlockSpec((BR, 1), lambda i: (i, 0)),
        pl.BlockSpec((1, D), lambda i: (0, 0)),
        pl.BlockSpec((1, D), lambda i: (0, 0)),
        pl.BlockSpec((1, 1), lambda i: (0, 0)),
        pl.BlockSpec((BR, 1), lambda i: (i, 0)),
    ],
    out_specs=pl.BlockSpec((1, G), lambda i: (0, 0)),
    out_shape=jax.ShapeDtypeStruct((1, G), jnp.float32),
    scratch_shapes=[
        pltpu.VMEM((1, G), jnp.float32),
        pltpu.VMEM((1, G), jnp.float32),
    ],
)


def kernel(x, edge_index, batch, W1, b1, W2, b2, Wfc, bfc):
    ei = edge_index.astype(jnp.int32)
    src3 = ei[0].reshape(NT, NCH, K)
    dst3 = ei[1].reshape(NT, NCH, K)
    pad = EPTP - EPT
    src_p = jnp.concatenate(
        [ei[0].reshape(NT, EPT), jnp.zeros((NT, pad), jnp.int32)], axis=1)
    dst_pad = N + (jnp.arange(pad, dtype=jnp.int32) % (NPAD - N))
    dst_p = jnp.concatenate(
        [ei[1].reshape(NT, EPT),
         jnp.broadcast_to(dst_pad, (NT, pad))], axis=1)
    src4 = src_p.reshape(NT, NPH, PC, KE)
    dst4 = dst_p.reshape(NT, NPH, PC, KE)
    batch2 = batch.astype(jnp.int32).reshape(N, 1)

    ones1 = jnp.ones((K,), jnp.float32)
    zeros1 = jnp.zeros((16 * NPAD,), jnp.float32)
    zerosD = jnp.zeros((NPAD, D), jnp.float32)

    tile_off = (jnp.arange(NT, dtype=jnp.int32) % 16)[:, None, None] * NPAD
    deg_parts = _deg_kernel(dst3 + tile_off, ones1, zeros1)
    degT = deg_parts.reshape(NT, NPAD)[:, :N].T

    hs1, dinv = _mm1(x, W1, degT)
    acc1 = _edge_kernel(hs1, src4, dst4, zerosD)
    hs2 = _mid(acc1, hs1, dinv, b1.reshape(1, D), W2)
    acc2 = _edge_kernel(hs2, src4, dst4, zerosD)
    out = _fin(acc2, hs2, dinv, b2.reshape(1, D),
               Wfc.reshape(1, D), bfc.reshape(1, 1), batch2)
    return out.reshape(G)


# speedup vs baseline: 2.5160x; 2.5160x over previous
import functools

import jax
import jax.numpy as jnp
from jax import lax
from jax.experimental import pallas as pl
from jax.experimental.pallas import tpu as pltpu
from jax.experimental.pallas import tpu_sc as plsc

N = 10000
E = 320000
D = 128
G = 64
NT = 32
EPT = E // NT
K = 80
NCH = EPT // K
NPAD = 10240
RPT = NPAD // 16
BR = 2000
GRID = N // BR

_mesh = plsc.VectorSubcoreMesh(core_axis_name="c", subcore_axis_name="s")


@functools.partial(
    pl.kernel,
    mesh=_mesh,
    out_type=jax.ShapeDtypeStruct((2, 16 * NPAD), jnp.float32),
    scratch_types=[
        pltpu.VMEM((NCH, K), jnp.int32),
        pltpu.VMEM((K,), jnp.float32),
        pltpu.VMEM_SHARED((16 * NPAD,), jnp.float32),
    ],
)
def _deg_kernel(dst_hbm, ones_hbm, zeros_hbm, out_hbm, dst_v, ones_v, acc_sh):
    cid = lax.axis_index("c")
    sid = lax.axis_index("s")
    wid = cid * 16 + sid
    off = pl.multiple_of(sid * NPAD, 8)
    pltpu.sync_copy(dst_hbm.at[wid], dst_v)
    pltpu.sync_copy(ones_hbm, ones_v)
    pltpu.sync_copy(zeros_hbm.at[pl.ds(off, NPAD)],
                    acc_sh.at[pl.ds(off, NPAD)])
    plsc.subcore_barrier()

    def body(c, carry):
        pltpu.sync_copy(ones_v, acc_sh.at[dst_v.at[c]], add=True)
        return carry

    lax.fori_loop(0, NCH, body, 0)
    plsc.subcore_barrier()
    pltpu.sync_copy(acc_sh.at[pl.ds(off, NPAD)],
                    out_hbm.at[cid, pl.ds(off, NPAD)])


NPH = 5
PC = NCH // NPH
PPAIR = PC // 2


@functools.partial(
    pl.kernel,
    mesh=_mesh,
    out_type=jax.ShapeDtypeStruct((2, NPAD, D), jnp.float32),
    scratch_types=[
        pltpu.VMEM((PC, K), jnp.int32),
        pltpu.VMEM((PC, K), jnp.int32),
        pltpu.VMEM((K, D), jnp.float32),
        pltpu.VMEM((K, D), jnp.float32),
        pltpu.VMEM_SHARED((NPAD, D), jnp.float32),
        pltpu.SemaphoreType.DMA,
        pltpu.SemaphoreType.DMA,
    ],
)
def _edge_kernel(table_hbm, src_hbm, dst_hbm, zeros_hbm, out_hbm,
                 src_v, dst_v, rows_a, rows_b, acc_sh, sem_a, sem_b):
    cid = lax.axis_index("c")
    sid = lax.axis_index("s")
    wid = cid * 16 + sid
    pltpu.sync_copy(zeros_hbm.at[pl.ds(sid * RPT, RPT)],
                    acc_sh.at[pl.ds(sid * RPT, RPT)])
    plsc.subcore_barrier()

    def start(c, buf, sem):
        pltpu.async_copy(table_hbm.at[src_v.at[c]], buf, sem)

    def wait(c, buf, sem):
        pltpu.make_async_copy(table_hbm.at[src_v.at[c]], buf, sem).wait()

    def phase(p, carry):
        pltpu.sync_copy(src_hbm.at[wid, p], src_v)
        pltpu.sync_copy(dst_hbm.at[wid, p], dst_v)
        start(0, rows_a, sem_a)

        def body(i, carry2):
            c0 = 2 * i
            c1 = c0 + 1
            start(c1, rows_b, sem_b)
            wait(c0, rows_a, sem_a)
            pltpu.sync_copy(rows_a, acc_sh.at[dst_v.at[c0]], add=True)

            @pl.when(i < PPAIR - 1)
            def _():
                start(c0 + 2, rows_a, sem_a)

            wait(c1, rows_b, sem_b)
            pltpu.sync_copy(rows_b, acc_sh.at[dst_v.at[c1]], add=True)
            return carry2

        lax.fori_loop(0, PPAIR, body, 0)
        start(PC - 1, rows_a, sem_a)
        wait(PC - 1, rows_a, sem_a)
        pltpu.sync_copy(rows_a, acc_sh.at[dst_v.at[PC - 1]], add=True)
        return carry

    lax.fori_loop(0, NPH, phase, 0)
    plsc.subcore_barrier()
    pltpu.sync_copy(acc_sh.at[pl.ds(sid * RPT, RPT)],
                    out_hbm.at[cid, pl.ds(sid * RPT, RPT)])


_HI = jax.lax.Precision.HIGHEST


def _mmh_body(x_ref, w_ref, h_ref):
    h_ref[...] = jnp.dot(x_ref[...], w_ref[...],
                         preferred_element_type=jnp.float32, precision=_HI)


def _scale_body(h_ref, dg_ref, hs_ref, dinv_ref):
    deg = jnp.sum(dg_ref[...], axis=1, keepdims=True) + 1.0
    dinv = lax.rsqrt(deg)
    hs_ref[...] = h_ref[...] * dinv
    dinv_ref[...] = dinv


def _mid_body(acc_ref, hs_ref, dinv_ref, b_ref, w_ref, out_ref):
    dinv = dinv_ref[...]
    m = (acc_ref[0] + acc_ref[1] + hs_ref[...]) * dinv + b_ref[...]
    m = jnp.maximum(m, 0.0)
    out_ref[...] = jnp.dot(m, w_ref[...], preferred_element_type=jnp.float32,
                           precision=_HI) * dinv


def _fin_body(acc_ref, hs_ref, dinv_ref, b_ref, wfc_ref, bfc_ref, batch_ref,
              out_ref, sums_ref, cnts_ref):
    i = pl.program_id(0)
    m = (acc_ref[0] + acc_ref[1] + hs_ref[...]) * dinv_ref[...] + b_ref[...]
    m = jnp.maximum(m, 0.0)
    y = jnp.sum(m * wfc_ref[...], axis=1, keepdims=True)
    oh = (batch_ref[...] == lax.broadcasted_iota(jnp.int32, (BR, G), 1))
    oh = oh.astype(jnp.float32)
    s = jnp.sum(oh * y, axis=0, keepdims=True)
    c = jnp.sum(oh, axis=0, keepdims=True)

    @pl.when(i == 0)
    def _():
        sums_ref[...] = s
        cnts_ref[...] = c

    @pl.when(i > 0)
    def _():
        sums_ref[...] += s
        cnts_ref[...] += c

    @pl.when(i == GRID - 1)
    def _():
        out_ref[...] = (sums_ref[...] / jnp.maximum(cnts_ref[...], 1.0)
                        + bfc_ref[...])


_mmh = pl.pallas_call(
    _mmh_body,
    grid=(GRID,),
    in_specs=[
        pl.BlockSpec((BR, D), lambda i: (i, 0)),
        pl.BlockSpec((D, D), lambda i: (0, 0)),
    ],
    out_specs=pl.BlockSpec((BR, D), lambda i: (i, 0)),
    out_shape=jax.ShapeDtypeStruct((N, D), jnp.float32),
)

_scale = pl.pallas_call(
    _scale_body,
    grid=(GRID,),
    in_specs=[
        pl.BlockSpec((BR, D), lambda i: (i, 0)),
        pl.BlockSpec((BR, NT), lambda i: (i, 0)),
    ],
    out_specs=[
        pl.BlockSpec((BR, D), lambda i: (i, 0)),
        pl.BlockSpec((BR, 1), lambda i: (i, 0)),
    ],
    out_shape=[
        jax.ShapeDtypeStruct((N, D), jnp.float32),
        jax.ShapeDtypeStruct((N, 1), jnp.float32),
    ],
)

_mid = pl.pallas_call(
    _mid_body,
    grid=(GRID,),
    in_specs=[
        pl.BlockSpec((2, BR, D), lambda i: (0, i, 0)),
        pl.BlockSpec((BR, D), lambda i: (i, 0)),
        pl.BlockSpec((BR, 1), lambda i: (i, 0)),
        pl.BlockSpec((1, D), lambda i: (0, 0)),
        pl.BlockSpec((D, D), lambda i: (0, 0)),
    ],
    out_specs=pl.BlockSpec((BR, D), lambda i: (i, 0)),
    out_shape=jax.ShapeDtypeStruct((N, D), jnp.float32),
)

_fin = pl.pallas_call(
    _fin_body,
    grid=(GRID,),
    in_specs=[
        pl.BlockSpec((2, BR, D), lambda i: (0, i, 0)),
        pl.BlockSpec((BR, D), lambda i: (i, 0)),
        pl.BlockSpec((BR, 1), lambda i: (i, 0)),
        pl.BlockSpec((1, D), lambda i: (0, 0)),
        pl.BlockSpec((1, D), lambda i: (0, 0)),
        pl.BlockSpec((1, 1), lambda i: (0, 0)),
        pl.BlockSpec((BR, 1), lambda i: (i, 0)),
    ],
    out_specs=pl.BlockSpec((1, G), lambda i: (0, 0)),
    out_shape=jax.ShapeDtypeStruct((1, G), jnp.float32),
    scratch_shapes=[
        pltpu.VMEM((1, G), jnp.float32),
        pltpu.VMEM((1, G), jnp.float32),
    ],
)


def kernel(x, edge_index, batch, W1, b1, W2, b2, Wfc, bfc):
    ei = edge_index.astype(jnp.int32)
    src3 = ei[0].reshape(NT, NCH, K)
    dst3 = ei[1].reshape(NT, NCH, K)
    src4 = ei[0].reshape(NT, NPH, PC, K)
    dst4 = ei[1].reshape(NT, NPH, PC, K)
    batch2 = batch.astype(jnp.int32).reshape(N, 1)

    ones1 = jnp.ones((K,), jnp.float32)
    zeros1 = jnp.zeros((16 * NPAD,), jnp.float32)
    zerosD = jnp.zeros((NPAD, D), jnp.float32)

    tile_off = (jnp.arange(NT, dtype=jnp.int32) % 16)[:, None, None] * NPAD
    h1 = _mmh(x, W1)
    deg_parts = _deg_kernel(dst3 + tile_off, ones1, zeros1)
    degT = deg_parts.reshape(NT, NPAD)[:, :N].T

    hs1, dinv = _scale(h1, degT)
    acc1 = _edge_kernel(hs1, src4, dst4, zerosD)
    hs2 = _mid(acc1, hs1, dinv, b1.reshape(1, D), W2)
    acc2 = _edge_kernel(hs2, src4, dst4, zerosD)
    out = _fin(acc2, hs2, dinv, b2.reshape(1, D),
               Wfc.reshape(1, D), bfc.reshape(1, 1), batch2)
    return out.reshape(G)
